# baseline (speedup 1.0000x reference)
import functools

import jax
import jax.numpy as jnp
from jax import lax
from jax.experimental import pallas as pl
from jax.experimental.pallas import tpu as pltpu
from jax.experimental.pallas import tpu_sc as plsc

BN_EPS = 1e-3

NC = 2
NS = 16
NW = NC * NS
CHUNK = 128


def _sc_gather(table, idx2, n_rows, embed):
    b_per_w = idx2.shape[1]
    unroll = 16
    mesh = plsc.VectorSubcoreMesh(core_axis_name="c", subcore_axis_name="s")

    @functools.partial(
        pl.kernel,
        mesh=mesh,
        out_type=jax.ShapeDtypeStruct((n_rows, embed), jnp.float32),
        scratch_types=[
            pltpu.VMEM((b_per_w,), jnp.int32),
            pltpu.SemaphoreType.DMA,
        ],
    )
    def gather_kernel(table_hbm, idx_hbm, out_hbm, idx_s, sem):
        wid = lax.axis_index("s") * NC + lax.axis_index("c")
        base = wid * b_per_w
        pltpu.sync_copy(idx_hbm.at[wid], idx_s)

        @pl.loop(0, b_per_w, step=unroll)
        def _(j):
            v = idx_s[pl.ds(j, unroll)]
            for u in range(unroll):
                pltpu.make_async_copy(
                    table_hbm.at[v[u]],
                    out_hbm.at[base + j + u],
                    sem,
                ).start()

        pltpu.make_async_copy(
            table_hbm.at[pl.ds(0, b_per_w)],
            out_hbm.at[pl.ds(base, b_per_w)],
            sem,
        ).wait()

    return gather_kernel(table, idx2)


def _mlp_body(x_ref, w1_ref, b1_ref, g_ref, be_ref, mm_ref, mv_ref,
              w2_ref, b2_ref, o_ref):
    h = jnp.dot(x_ref[...], w1_ref[...], preferred_element_type=jnp.float32)
    h = jnp.maximum(h + b1_ref[...], 0.0)
    s = g_ref[...] * lax.rsqrt(mv_ref[...] + BN_EPS)
    t = be_ref[...] - s * mm_ref[...]
    h = h * s + t
    o_ref[...] = (
        jnp.dot(h, w2_ref[...], preferred_element_type=jnp.float32)
        + b2_ref[...]
    )


def _tc_mlp(x, W1, b1, gamma, beta, mmean, mvar, W2, b2, block_m):
    n, embed = x.shape
    hdim = W2.shape[1]
    row = lambda v: v.reshape(1, -1)
    vec_spec = pl.BlockSpec((1, hdim), lambda i: (0, 0))
    return pl.pallas_call(
        _mlp_body,
        grid=(n // block_m,),
        in_specs=[
            pl.BlockSpec((block_m, embed), lambda i: (i, 0)),
            pl.BlockSpec((embed, hdim), lambda i: (0, 0)),
            vec_spec, vec_spec, vec_spec, vec_spec, vec_spec,
            pl.BlockSpec((hdim, hdim), lambda i: (0, 0)),
            vec_spec,
        ],
        out_specs=pl.BlockSpec((block_m, hdim), lambda i: (i, 0)),
        out_shape=jax.ShapeDtypeStruct((n, hdim), jnp.float32),
    )(x, W1, row(b1), row(gamma), row(beta), row(mmean), row(mvar),
      W2, row(b2))


def kernel(anchor, positive, negative, emb_table, W1, b1, gamma, beta,
           moving_mean, moving_var, W2, b2):
    b = anchor.shape[0]
    nb = 3 * b
    idx = jnp.concatenate([anchor, positive, negative]).astype(jnp.int32)
    idx2 = idx.reshape(NW, nb // NW)
    gathered = _sc_gather(emb_table, idx2, nb, emb_table.shape[1])
    out = _tc_mlp(gathered, W1, b1, gamma, beta, moving_mean, moving_var,
                  W2, b2, block_m=2048)
    return (out[:b], out[b:2 * b], out[2 * b:])


# speedup vs baseline: 1.4781x; 1.4781x over previous
import functools

import jax
import jax.numpy as jnp
from jax import lax
from jax.experimental import pallas as pl
from jax.experimental.pallas import tpu as pltpu
from jax.experimental.pallas import tpu_sc as plsc

BN_EPS = 1e-3

NC = 2
NS = 16
NW = NC * NS


def _sc_gather(table, idx2, n_rows, embed):
    b_per_w = idx2.shape[1]
    K = 128
    nbatch = b_per_w // K
    pair = 2 * embed
    mesh = plsc.VectorSubcoreMesh(core_axis_name="c", subcore_axis_name="s")
    table2 = table.reshape(table.shape[0] // 2, pair)

    @functools.partial(
        pl.kernel,
        mesh=mesh,
        compiler_params=pltpu.CompilerParams(needs_layout_passes=False),
        out_type=jax.ShapeDtypeStruct((n_rows, embed), jnp.float32),
        scratch_types=[
            pltpu.VMEM((b_per_w,), jnp.int32),
            pltpu.VMEM((b_per_w,), jnp.int32),
            pltpu.VMEM((2, K, pair), jnp.float32),
            pltpu.VMEM((2, K, embed), jnp.float32),
            pltpu.SemaphoreType.DMA,
            pltpu.SemaphoreType.DMA,
        ],
    )
    def gather_kernel(table_hbm, idx_hbm, out_hbm, idx_v, pidx_v,
                      pairs_v, rows_v, gsem, wsem):
        wid = lax.axis_index("s") * NC + lax.axis_index("c")
        wbase = wid * b_per_w
        pltpu.sync_copy(idx_hbm.at[wid], idx_v)

        @pl.loop(0, b_per_w, step=16)
        def _(j):
            pidx_v[pl.ds(j, 16)] = lax.shift_right_logical(
                idx_v[pl.ds(j, 16)], 1)

        def start_gather(b, buf):
            pltpu.make_async_copy(
                table_hbm.at[pidx_v.at[pl.ds(b * K, K)]],
                pairs_v.at[buf],
                gsem,
            ).start()

        def wait_gather(b, buf):
            pltpu.make_async_copy(
                table_hbm.at[pidx_v.at[pl.ds(b * K, K)]],
                pairs_v.at[buf],
                gsem,
            ).wait()

        def do_batch(b, buf):
            wait_gather(b, buf)

            @pl.when(b >= 2)
            def _():
                pltpu.make_async_copy(
                    rows_v.at[buf],
                    out_hbm.at[pl.ds(wbase + (b - 2) * K, K)],
                    wsem,
                ).wait()

            blk = pairs_v.at[buf]
            rows = rows_v.at[buf]
            lane = lax.iota(jnp.int32, 16)
            for g in range(K // 16):
                idxg = idx_v[pl.ds(b * K + g * 16, 16)]
                coff = lax.bitwise_and(idxg, jnp.int32(1)) * jnp.int32(embed)
                kvec = lane + jnp.int32(g * 16)
                for c in range(embed):
                    cvec = jnp.full((16,), c, jnp.int32)
                    vals = plsc.load_gather(blk, [kvec, cvec + coff])
                    plsc.store_scatter(rows, [kvec, cvec], vals)

            @pl.when(b + 2 < nbatch)
            def _():
                start_gather(b + 2, buf)

            pltpu.make_async_copy(
                rows,
                out_hbm.at[pl.ds(wbase + b * K, K)],
                wsem,
            ).start()

        start_gather(0, 0)
        start_gather(1, 1)

        @pl.loop(0, nbatch, step=2)
        def _(b):
            do_batch(b, 0)
            do_batch(b + 1, 1)

        for buf in range(2):
            pltpu.make_async_copy(
                rows_v.at[buf],
                out_hbm.at[pl.ds(wbase + (nbatch - 2 + buf) * K, K)],
                wsem,
            ).wait()

    return gather_kernel(table2, idx2)


def _mlp_body(x_ref, w1_ref, b1_ref, g_ref, be_ref, mm_ref, mv_ref,
              w2_ref, b2_ref, o_ref):
    h = jnp.dot(x_ref[...], w1_ref[...], preferred_element_type=jnp.float32)
    h = jnp.maximum(h + b1_ref[...], 0.0)
    s = g_ref[...] * lax.rsqrt(mv_ref[...] + BN_EPS)
    t = be_ref[...] - s * mm_ref[...]
    h = h * s + t
    o_ref[...] = (
        jnp.dot(h, w2_ref[...], preferred_element_type=jnp.float32)
        + b2_ref[...]
    )


def _tc_mlp(x, W1, b1, gamma, beta, mmean, mvar, W2, b2, block_m):
    n, embed = x.shape
    hdim = W2.shape[1]
    row = lambda v: v.reshape(1, -1)
    vec_spec = pl.BlockSpec((1, hdim), lambda i: (0, 0))
    return pl.pallas_call(
        _mlp_body,
        grid=(n // block_m,),
        in_specs=[
            pl.BlockSpec((block_m, embed), lambda i: (i, 0)),
            pl.BlockSpec((embed, hdim), lambda i: (0, 0)),
            vec_spec, vec_spec, vec_spec, vec_spec, vec_spec,
            pl.BlockSpec((hdim, hdim), lambda i: (0, 0)),
            vec_spec,
        ],
        out_specs=pl.BlockSpec((block_m, hdim), lambda i: (i, 0)),
        out_shape=jax.ShapeDtypeStruct((n, hdim), jnp.float32),
    )(x, W1, row(b1), row(gamma), row(beta), row(mmean), row(mvar),
      W2, row(b2))


def kernel(anchor, positive, negative, emb_table, W1, b1, gamma, beta,
           moving_mean, moving_var, W2, b2):
    b = anchor.shape[0]
    nb = 3 * b
    idx = jnp.concatenate([anchor, positive, negative]).astype(jnp.int32)
    idx2 = idx.reshape(NW, nb // NW)
    gathered = _sc_gather(emb_table, idx2, nb, emb_table.shape[1])
    out = _tc_mlp(gathered, W1, b1, gamma, beta, moving_mean, moving_var,
                  W2, b2, block_m=2048)
    return (out[:b], out[b:2 * b], out[2 * b:])
